# Initial kernel scaffold; baseline (speedup 1.0000x reference)
import jax
import jax.numpy as jnp
from jax.experimental import pallas as pl


def kernel(entity_embed, user_embed, relation_emb, W1, W2, edge_index, edge_type, ui_item_idx, ui_user_idx):
    raise NotImplementedError("write your pallas kernel here")


# speedup vs baseline: 3.1543x; 3.1543x over previous
import dataclasses
import functools

import jax
import jax.numpy as jnp
from jax import lax
from jax.experimental import pallas as pl
from jax.experimental.pallas import tpu as pltpu
from jax.experimental.pallas import tpu_sc as plsc

N_ENT = 10000
N_ITEMS = 4000
N_USERS = 6000
N_REL = 8
D = 128
E = 320000
E_UI = 120000

NW = 32
CH = 128

E_PAD = 323584
NCH_E = (E_PAD // NW) // CH
EG_PAD = 2 * E_PAD
NCH_G = (EG_PAD // NW) // CH
EU_PAD = 122880
NCH_U = (EU_PAD // NW) // CH

NA = 10240
NU = 6144
NI = 4096


@functools.cache
def _mesh():
    return plsc.VectorSubcoreMesh(core_axis_name="c", subcore_axis_name="s")


def _pad1(x, n, fill):
    return jnp.concatenate([x, jnp.full((n - x.shape[0],), fill, x.dtype)])


def _t0_body(ent_ref, usr_ref, p_ref, ud_ref):
    u = ent_ref[...]
    u2 = jnp.sum(u * u, axis=1, keepdims=True)
    n = jnp.clip(jnp.sqrt(jnp.clip(u2, 1e-15)), 1e-7)
    p_ref[...] = jnp.tanh(n) * u / n
    v = usr_ref[...]
    v2 = jnp.sum(v * v, axis=1, keepdims=True)
    ud_ref[...] = v / (v2 + 1e-6)


def _t0(entity_embed, user_embed):
    return pl.pallas_call(
        _t0_body,
        out_shape=(
            jax.ShapeDtypeStruct((N_ENT, D), jnp.float32),
            jax.ShapeDtypeStruct((N_USERS, D), jnp.float32),
        ),
    )(entity_embed, user_embed)


def _hist_zero(h_ref, n):
    z = jnp.zeros((16,), jnp.float32)

    @pl.loop(0, n // 16)
    def _(j):
        h_ref[pl.ds(j * 16, 16)] = z


def _hist_accum(idx_hbm, h_ref, idx_v, isem, wid, nch):
    ones16 = jnp.ones((16,), jnp.float32)

    @pl.loop(0, nch)
    def _(c):
        base = wid * (nch * CH) + c * CH
        pltpu.async_copy(idx_hbm.at[pl.ds(base, CH)], idx_v, isem).wait()

        @pl.loop(0, CH // 16)
        def _(k):
            v = idx_v[pl.ds(k * 16, 16)]
            plsc.addupdate_scatter(h_ref, [v], ones16)


def _s0_body(eidx_hbm, uidx_hbm, iidx_hbm, he_hbm, hu_hbm, hi_hbm,
             h_e, h_u, h_i, idx_v, isem):
    wid = lax.axis_index("s") * 2 + lax.axis_index("c")
    _hist_zero(h_e, NA)
    _hist_zero(h_u, NU)
    _hist_zero(h_i, NI)
    _hist_accum(eidx_hbm, h_e, idx_v, isem, wid, NCH_E)
    _hist_accum(uidx_hbm, h_u, idx_v, isem, wid, NCH_U)
    _hist_accum(iidx_hbm, h_i, idx_v, isem, wid, NCH_U)
    pltpu.sync_copy(h_e, he_hbm.at[wid])
    pltpu.sync_copy(h_u, hu_hbm.at[wid])
    pltpu.sync_copy(h_i, hi_hbm.at[wid])


@functools.cache
def _s0():
    cp = pltpu.CompilerParams()
    if "needs_layout_passes" in pltpu.CompilerParams.__dataclass_fields__:
        cp = dataclasses.replace(cp, needs_layout_passes=False)
    return pl.kernel(
        _s0_body,
        compiler_params=cp,
        out_type=(
            jax.ShapeDtypeStruct((NW, NA), jnp.float32),
            jax.ShapeDtypeStruct((NW, NU), jnp.float32),
            jax.ShapeDtypeStruct((NW, NI), jnp.float32),
        ),
        mesh=_mesh(),
        scratch_types=[
            pltpu.VMEM((NA,), jnp.float32),
            pltpu.VMEM((NU,), jnp.float32),
            pltpu.VMEM((NI,), jnp.float32),
            pltpu.VMEM((CH,), jnp.int32),
            pltpu.SemaphoreType.DMA,
        ],
    )


def _colsum(h):
    return lax.dot_general(h, jnp.ones((NW, 1), jnp.float32),
                           (((0,), (0,)), ((), ())),
                           preferred_element_type=jnp.float32)


def _s1_body(tab_hbm, idx_hbm, out_hbm, idx_v, buf, sem, isem):
    wid = lax.axis_index("s") * 2 + lax.axis_index("c")
    pltpu.async_copy(idx_hbm.at[wid], idx_v, isem).wait()

    @pl.loop(0, NCH_G)
    def _(c):
        base = wid * (NCH_G * CH) + c * CH
        pltpu.async_copy(tab_hbm.at[idx_v.at[c]], buf, sem).wait()
        pltpu.async_copy(buf, out_hbm.at[pl.ds(base, CH)], sem).wait()


@functools.cache
def _s1():
    return pl.kernel(
        _s1_body,
        out_type=jax.ShapeDtypeStruct((EG_PAD, D), jnp.float32),
        mesh=_mesh(),
        scratch_types=[
            pltpu.VMEM((NCH_G, CH), jnp.int32),
            pltpu.VMEM((CH, D), jnp.float32),
            pltpu.SemaphoreType.DMA,
            pltpu.SemaphoreType.DMA,
        ],
    )


def _exp_co(u2, pu, pp, lam):
    nu = jnp.clip(jnp.sqrt(jnp.clip(u2, 1e-15)), 1e-7)
    s = jnp.tanh(lam * nu / 2.0) / nu
    y2 = s * s * u2
    xy = s * pu
    den = jnp.clip(1.0 + 2.0 * xy + pp * y2, 1e-15)
    return (1.0 + 2.0 * xy + y2) / den, (1.0 - pp) * s / den


def _t1_body(gs_ref, gd_ref, et_ref, rel_ref, tan_ref):
    p = gs_ref[...]
    d = gd_ref[...]
    et = et_ref[...]
    r = jnp.broadcast_to(rel_ref[0:1, :], p.shape)
    for k in range(1, N_REL):
        r = jnp.where(et == k, rel_ref[k:k + 1, :], r)

    pp = jnp.sum(p * p, axis=1, keepdims=True)
    dd = jnp.sum(d * d, axis=1, keepdims=True)
    rr = jnp.sum(r * r, axis=1, keepdims=True)
    pd = jnp.sum(p * d, axis=1, keepdims=True)
    pr = jnp.sum(p * r, axis=1, keepdims=True)
    dr = jnp.sum(d * r, axis=1, keepdims=True)

    lam = 2.0 / jnp.clip(1.0 - pp, 1e-7)
    c1, c2 = _exp_co(dd, pd, pp, lam)
    d1, d2 = _exp_co(rr, pr, pp, lam)

    a2 = c1 * c1 * pp + c2 * c2 * dd + 2.0 * c1 * c2 * pd
    b2 = d1 * d1 * pp + d2 * d2 * rr + 2.0 * d1 * d2 * pr
    ab = c1 * d1 * pp + c1 * d2 * pr + c2 * d1 * pd + c2 * d2 * dr

    den_m = jnp.clip(1.0 + 2.0 * ab + a2 * b2, 1e-15)
    e1 = (1.0 + 2.0 * ab + b2) / den_m
    e2 = (1.0 - a2) / den_m
    m1 = e1 * c1 + e2 * d1
    m2 = e1 * c2
    m3 = e2 * d2

    m2n = (m1 * m1 * pp + m2 * m2 * dd + m3 * m3 * rr
           + 2.0 * (m1 * m2 * pd + m1 * m3 * pr + m2 * m3 * dr))
    nm = jnp.sqrt(jnp.clip(m2n, 1e-15))
    maxn = 1.0 - 1e-5
    scale = jnp.where(nm > maxn, maxn / nm, 1.0)

    pm = m1 * pp + m2 * pd + m3 * pr
    y2p = scale * scale * m2n
    xyp = -scale * pm
    den_s = jnp.clip(1.0 + 2.0 * xyp + pp * y2p, 1e-15)
    amp = (1.0 + 2.0 * xyp + y2p) / den_s
    bmp = (1.0 - pp) * scale / den_s
    g1 = bmp * m1 - amp
    g2 = bmp * m2
    g3 = bmp * m3

    sub = g1 * p + g2 * d + g3 * r
    s2 = jnp.sum(sub * sub, axis=1, keepdims=True)
    ns = jnp.clip(jnp.sqrt(jnp.clip(s2, 1e-15)), 1e-7)
    nsc = jnp.clip(ns, -1.0 + 1e-7, 1.0 - 1e-7)
    at = 0.5 * jnp.log((1.0 + nsc) / (1.0 - nsc))
    coef = (2.0 / lam) * at / ns
    tan_ref[...] = coef * sub


_T1B = 512


def _t1(g_all, et_col, rel2):
    grid = E_PAD // _T1B
    return pl.pallas_call(
        _t1_body,
        grid=(grid,),
        in_specs=[
            pl.BlockSpec((_T1B, D), lambda i: (i, 0)),
            pl.BlockSpec((_T1B, D), lambda i: (i + grid, 0)),
            pl.BlockSpec((_T1B, 1), lambda i: (i, 0)),
            pl.BlockSpec((N_REL, D), lambda i: (0, 0)),
        ],
        out_specs=pl.BlockSpec((_T1B, D), lambda i: (i, 0)),
        out_shape=jax.ShapeDtypeStruct((E_PAD, D), jnp.float32),
    )(g_all, g_all, et_col, rel2)


def _zero_strips(sid, zr_hbm, acc_sh, buf, nrows):
    rows_per_tile = nrows // 16
    nstrip = rows_per_tile // CH
    pltpu.sync_copy(zr_hbm, buf)

    @pl.loop(0, nstrip)
    def _(j):
        pltpu.sync_copy(buf, acc_sh.at[pl.ds(sid * rows_per_tile + j * CH, CH)])


def _copyout_strips(cid, sid, acc_sh, sum_hbm, buf, nrows):
    rows_per_tile = nrows // 16
    nstrip = rows_per_tile // CH

    @pl.loop(0, nstrip)
    def _(j):
        base = sid * rows_per_tile + j * CH
        pltpu.sync_copy(acc_sh.at[pl.ds(base, CH)], buf)
        pltpu.sync_copy(buf, sum_hbm.at[pl.ds(cid * nrows + base, CH)])


def _s2_body(tan_hbm, idx_hbm, zr_hbm, sum_hbm, acc_sh, idx_v, buf, sem, isem):
    cid = lax.axis_index("c")
    sid = lax.axis_index("s")
    wid = sid * 2 + cid

    _zero_strips(sid, zr_hbm, acc_sh, buf, NA)
    plsc.subcore_barrier()

    @pl.loop(0, NCH_E)
    def _(c):
        base = wid * (NCH_E * CH) + c * CH
        cp = pltpu.async_copy(idx_hbm.at[pl.ds(base, CH)], idx_v, isem)
        pltpu.async_copy(tan_hbm.at[pl.ds(base, CH)], buf, sem).wait()
        cp.wait()
        pltpu.sync_copy(buf, acc_sh.at[idx_v], add=True)

    plsc.subcore_barrier()
    _copyout_strips(cid, sid, acc_sh, sum_hbm, buf, NA)


@functools.cache
def _s2():
    return pl.kernel(
        _s2_body,
        out_type=jax.ShapeDtypeStruct((2 * NA, D), jnp.float32),
        mesh=_mesh(),
        scratch_types=[
            pltpu.VMEM_SHARED((NA, D), jnp.float32),
            pltpu.VMEM((CH,), jnp.int32),
            pltpu.VMEM((CH, D), jnp.float32),
            pltpu.SemaphoreType.DMA,
            pltpu.SemaphoreType.DMA,
        ],
    )


def _t2_body(sum_ref, he_ref, out_ref):
    s = sum_ref[:NA] + sum_ref[NA:]
    c = jnp.maximum(_colsum(he_ref[...]), 1.0)
    out_ref[...] = s / c


def _t2(sum2, hist_e):
    return pl.pallas_call(
        _t2_body,
        out_shape=jax.ShapeDtypeStruct((NA, D), jnp.float32),
    )(sum2, hist_e)


def _s3_body(tab_hbm, gidx_hbm, sidx_hbm, zr_hbm, sum_hbm,
             acc_sh, gidx_v, sidx_v, buf, sem, isem, *, nrows):
    cid = lax.axis_index("c")
    sid = lax.axis_index("s")
    wid = sid * 2 + cid

    _zero_strips(sid, zr_hbm, acc_sh, buf, nrows)
    pltpu.async_copy(gidx_hbm.at[wid], gidx_v, isem).wait()
    pltpu.async_copy(sidx_hbm.at[wid], sidx_v, isem).wait()
    plsc.subcore_barrier()

    @pl.loop(0, NCH_U)
    def _(c):
        pltpu.async_copy(tab_hbm.at[gidx_v.at[c]], buf, sem).wait()
        pltpu.sync_copy(buf, acc_sh.at[sidx_v.at[c]], add=True)

    plsc.subcore_barrier()
    _copyout_strips(cid, sid, acc_sh, sum_hbm, buf, nrows)


@functools.cache
def _s3(nrows):
    return pl.kernel(
        functools.partial(_s3_body, nrows=nrows),
        out_type=jax.ShapeDtypeStruct((2 * nrows, D), jnp.float32),
        mesh=_mesh(),
        scratch_types=[
            pltpu.VMEM_SHARED((nrows, D), jnp.float32),
            pltpu.VMEM((NCH_U, CH), jnp.int32),
            pltpu.VMEM((NCH_U, CH), jnp.int32),
            pltpu.VMEM((CH, D), jnp.float32),
            pltpu.SemaphoreType.DMA,
            pltpu.SemaphoreType.DMA,
        ],
    )


def _t3u_body(sum_ref, hu_ref, u_ref):
    s = sum_ref[:N_USERS] + sum_ref[NU:NU + N_USERS]
    c = jnp.maximum(_colsum(hu_ref[...])[:N_USERS], 1.0)
    u_ref[...] = s / c


def _t3u(usum2, hist_u):
    return pl.pallas_call(
        _t3u_body,
        out_shape=jax.ShapeDtypeStruct((N_USERS, D), jnp.float32),
    )(usum2, hist_u)


def _t3i_body(sum_ref, hi_ref, oi_ref, w1_ref, w2_ref, fus_ref):
    oi = oi_ref[...]
    s = sum_ref[:N_ITEMS] + sum_ref[NI:NI + N_ITEMS]
    c = jnp.maximum(_colsum(hi_ref[...])[:N_ITEMS], 1.0)
    norm_i = jnp.sum(oi * oi, axis=1, keepdims=True)
    i_cf = (s / c) * norm_i
    dn = (((1,), (1,)), ((), ()))
    z = (lax.dot_general(oi, w1_ref[...], dn, preferred_element_type=jnp.float32)
         + lax.dot_general(i_cf, w2_ref[...], dn, preferred_element_type=jnp.float32))
    gi = jax.nn.sigmoid(z)
    fus_ref[...] = gi * oi + (1.0 - gi) * i_cf


def _t3i(isum2, hist_i, out_items, W1, W2):
    return pl.pallas_call(
        _t3i_body,
        out_shape=jax.ShapeDtypeStruct((N_ITEMS, D), jnp.float32),
    )(isum2, hist_i, out_items, W1, W2)


def kernel(entity_embed, user_embed, relation_emb, W1, W2,
           edge_index, edge_type, ui_item_idx, ui_user_idx):
    src = edge_index[0].astype(jnp.int32)
    dst = edge_index[1].astype(jnp.int32)
    et = edge_type.astype(jnp.int32)
    rel2 = relation_emb[2:2 + N_REL]

    P, user_div = _t0(entity_embed, user_embed)

    item_i = ui_item_idx.astype(jnp.int32)
    user_i = ui_user_idx.astype(jnp.int32)
    sidx = _pad1(src, E_PAD, N_ENT)
    sidx_u_flat = _pad1(user_i, EU_PAD, N_USERS)
    sidx_i_flat = _pad1(item_i, EU_PAD, N_ITEMS)
    hist_e, hist_u, hist_i = _s0()(sidx, sidx_u_flat, sidx_i_flat)

    tab = jnp.concatenate([P, entity_embed], axis=0)
    gidx = jnp.concatenate([
        _pad1(src, E_PAD, 0), _pad1(dst, E_PAD, 0) + N_ENT
    ]).reshape(NW, NCH_G, CH)
    g_all = _s1()(tab, gidx)

    et_col = _pad1(et, E_PAD, 0).reshape(E_PAD, 1)
    tan = _t1(g_all, et_col, rel2)

    zeros_rows = jnp.zeros((CH, D), jnp.float32)
    sum2 = _s2()(tan, sidx, zeros_rows)

    out_full = _t2(sum2, hist_e)
    out_items = out_full[:N_ITEMS]

    gidx_u = _pad1(item_i, EU_PAD, 0).reshape(NW, NCH_U, CH)
    sidx_u = sidx_u_flat.reshape(NW, NCH_U, CH)
    usum2 = _s3(NU)(out_items, gidx_u, sidx_u, zeros_rows)
    gidx_i = _pad1(user_i, EU_PAD, 0).reshape(NW, NCH_U, CH)
    sidx_i = sidx_i_flat.reshape(NW, NCH_U, CH)
    isum2 = _s3(NI)(user_div, gidx_i, sidx_i, zeros_rows)

    u = _t3u(usum2, hist_u)
    fusion = _t3i(isum2, hist_i, out_items, W1, W2)

    ret0 = jnp.concatenate([fusion, out_full[N_ITEMS:N_ENT]], axis=0)
    return (ret0, u, out_full[:N_ITEMS])
